# baseline (speedup 1.0000x reference)
import functools

import jax
import jax.numpy as jnp
from jax import lax
from jax.experimental import pallas as pl
from jax.experimental.pallas import tpu as pltpu
from jax.experimental.pallas import tpu_sc as plsc

N_ENT = 10000
N_USERS = 10000
N_ITEMS = 5000
E = 320000
NNZ = 500000
D = 128

SE = 10240
SI = 5120
NNZP = 500224
NC, NS = 2, 16
NW = NC * NS
IR = 4
BLK = 512


def _sc_mesh():
    return plsc.VectorSubcoreMesh(
        core_axis_name="c", subcore_axis_name="s",
        num_cores=NC, num_subcores=NS)


def _gather_call(table, idx2d):
    nch = idx2d.shape[0] // IR
    iters = -(-nch // NW)
    n_rows = idx2d.shape[0] * 128
    ch_rows = IR * 128

    @functools.partial(
        pl.kernel,
        out_type=jax.ShapeDtypeStruct((n_rows, D), jnp.float32),
        mesh=_sc_mesh(),
        scratch_types=[
            pltpu.VMEM((IR, 128), jnp.int32),
            pltpu.VMEM((ch_rows, D), jnp.float32),
            pltpu.SemaphoreType.DMA,
        ],
    )
    def k(tab, idx, out, idx_v, rows_v, sem):
        w = lax.axis_index("s") * NC + lax.axis_index("c")

        def body(i, carry):
            ch = w + i * NW

            @pl.when(ch < nch)
            def _():
                pltpu.sync_copy(idx.at[pl.ds(ch * IR, IR)], idx_v)
                descs = [
                    pltpu.async_copy(tab.at[idx_v.at[j]],
                                     rows_v.at[pl.ds(j * 128, 128)], sem)
                    for j in range(IR)
                ]
                for d in descs:
                    d.wait()
                pltpu.sync_copy(rows_v, out.at[pl.ds(ch * ch_rows, ch_rows)])

            return carry

        lax.fori_loop(0, iters, body, 0)

    return k(table, idx2d)


def _gather_scatter_call(table, gidx2d, sidx2d, zeros, S, ir=2):
    ch_rows = ir * 128
    nch = gidx2d.shape[0] // ir
    iters = -(-nch // NW)
    rpt = S // NS

    @functools.partial(
        pl.kernel,
        out_type=jax.ShapeDtypeStruct((NC * S, D), jnp.float32),
        mesh=_sc_mesh(),
        scratch_types=[
            pltpu.VMEM((ir, 128), jnp.int32),
            pltpu.VMEM((ir, 128), jnp.int32),
            pltpu.VMEM((ch_rows, D), jnp.float32),
            pltpu.VMEM_SHARED((S, D), jnp.float32),
            pltpu.SemaphoreType.DMA,
        ],
    )
    def k(tab, gidx, sidx, zz, out, gi_v, si_v, rows_v, acc, sem):
        c = lax.axis_index("c")
        s = lax.axis_index("s")
        w = s * NC + c
        pltpu.sync_copy(zz.at[pl.ds(s * rpt, rpt)], acc.at[pl.ds(s * rpt, rpt)])
        plsc.subcore_barrier()

        def body(i, carry):
            ch = w + i * NW

            @pl.when(ch < nch)
            def _():
                pltpu.sync_copy(gidx.at[pl.ds(ch * ir, ir)], gi_v)
                pltpu.sync_copy(sidx.at[pl.ds(ch * ir, ir)], si_v)
                descs = [
                    pltpu.async_copy(tab.at[gi_v.at[j]],
                                     rows_v.at[pl.ds(j * 128, 128)], sem)
                    for j in range(ir)
                ]
                for d in descs:
                    d.wait()
                for j in range(ir):
                    pltpu.sync_copy(rows_v.at[pl.ds(j * 128, 128)],
                                    acc.at[si_v.at[j]], add=True)

            return carry

        lax.fori_loop(0, iters, body, 0)
        plsc.subcore_barrier()
        pltpu.sync_copy(acc.at[pl.ds(s * rpt, rpt)],
                        out.at[pl.ds(c * S + s * rpt, rpt)])

    return k(table, gidx2d, sidx2d, zeros)


def _mob(x, y, x2, y2, xy):
    num = (1.0 + 2.0 * xy + y2) * x + (1.0 - x2) * y
    den = 1.0 + 2.0 * xy + x2 * y2
    return num / jnp.clip(den, 1e-15, None)


def _edge_body(h_ref, t_ref, et_ref, w_ref, o_ref):
    h = h_ref[...]
    t = t_ref[...]
    W = w_ref[...]
    rel = et_ref[0, 0, :] - 1

    r = jnp.zeros_like(h)
    for k in range(10):
        r = r + jnp.where(rel == k, 1.0, 0.0)[:, None] * W[k][None, :]

    def rsum(x, y):
        p = x * y
        v = p[:, 0:8]
        for g in range(1, 16):
            v = v + p[:, 8 * g:8 * g + 8]
        v = v[:, 0:4] + v[:, 4:8]
        v = v[:, 0:2] + v[:, 2:4]
        return v[:, 0:1] + v[:, 1:2]

    def nrm(x):
        return jnp.clip(jnp.sqrt(rsum(x, x)), 1e-15, None)

    hn = nrm(h)
    hh = jnp.tanh(hn) * h / hn
    hh2 = rsum(hh, hh)
    lam = 2.0 / jnp.clip(1.0 - hh2, 1e-15, None)

    tn = nrm(t)
    u = jnp.tanh(lam * tn / 2.0) * t / tn
    ht = _mob(hh, u, hh2, rsum(u, u), rsum(hh, u))

    rn = nrm(r)
    v = jnp.tanh(lam * rn / 2.0) * r / rn
    hr = _mob(hh, v, hh2, rsum(v, v), rsum(hh, v))

    a2 = rsum(ht, ht)
    b2 = rsum(hr, hr)
    ab = rsum(ht, hr)
    num_r = (1.0 + 2.0 * ab + b2) * ht + (1.0 - a2) * hr
    den_r = jnp.clip(1.0 + 2.0 * ab + a2 * b2, 1e-15, None)
    res = num_r / den_r
    n = nrm(res)
    mx = 1.0 - 1e-5
    res = jnp.where(n > mx, num_r / (den_r * n) * mx, res)

    sub = _mob(-hh, res, hh2, rsum(res, res), rsum(-hh, res))
    sn = nrm(sub)
    z = jnp.clip(sn, -1.0 + 1e-7, 1.0 - 1e-7)
    art = 0.5 * (jnp.log1p(z) - jnp.log1p(-z))
    tl = jnp.clip(1.0 - hh2, 1e-15, None)
    o_ref[...] = (tl * art) * sub / sn


def _edge_call(htr, et3, w_pad):
    nb = E // BLK
    return pl.pallas_call(
        _edge_body,
        grid=(nb,),
        in_specs=[
            pl.BlockSpec((BLK, D), lambda i: (i, 0)),
            pl.BlockSpec((BLK, D), lambda i: (i + nb, 0)),
            pl.BlockSpec((1, 1, BLK), lambda i: (i, 0, 0)),
            pl.BlockSpec((16, D), lambda i: (0, 0)),
        ],
        out_specs=pl.BlockSpec((BLK, D), lambda i: (i, 0)),
        out_shape=jax.ShapeDtypeStruct((E, D), jnp.float32),
    )(htr, htr, et3, w_pad)


def _l2n(x):
    return x / jnp.clip(jnp.linalg.norm(x, axis=-1, keepdims=True), 1e-12, None)


def _dr_norm(user_emb, entity_emb, item_emb_cf):
    for i in range(3):
        entity_emb = _l2n(entity_emb)
        user_emb = _l2n(user_emb)
        item_emb_cf = _l2n(item_emb_cf)
        if i == 0:
            ea, ua, ia = entity_emb, user_emb, item_emb_cf
        else:
            ea = ea + entity_emb
            ua = ua + user_emb
            ia = ia + item_emb_cf
    return ea, ua, ia


def kernel(user_emb, entity_emb, item_emb_cf, edge_index, edge_type,
           interact_indices, interact_values, relation_weight):
    f32 = jnp.float32
    ei = edge_index.astype(jnp.int32)
    head = ei[0]
    tail = ei[1]
    ht_idx = jnp.concatenate([head, tail]).reshape(2 * E // 128, 128)
    et3 = edge_type.astype(jnp.int32).reshape(E // BLK, 1, BLK)

    mr = interact_indices[0].astype(jnp.int32)
    mc = interact_indices[1].astype(jnp.int32)
    pad = NNZP - NNZ
    mr_g = jnp.concatenate([mr, jnp.zeros((pad,), jnp.int32)]).reshape(-1, 128)
    mc_g = jnp.concatenate([mc, jnp.zeros((pad,), jnp.int32)]).reshape(-1, 128)
    mr_s = jnp.concatenate(
        [mr, jnp.full((pad,), N_USERS, jnp.int32)]).reshape(-1, 128)
    mc_s = jnp.concatenate(
        [mc, jnp.full((pad,), N_ITEMS, jnp.int32)]).reshape(-1, 128)

    w_pad = jnp.zeros((16, D), f32).at[:relation_weight.shape[0]].set(
        relation_weight)
    z_se = jnp.zeros((SE, D), f32)
    z_si = jnp.zeros((SI, D), f32)

    def pad_rows(x, S):
        return jnp.zeros((S, D), f32).at[:x.shape[0]].set(x)

    def _aggregate(ent_t, usr_t, itm_t):
        htr = _gather_call(pad_rows(ent_t, SE), ht_idx)
        res = _edge_call(htr, et3, w_pad)
        s = jax.ops.segment_sum(res, head, num_segments=N_ENT)
        c = jax.ops.segment_sum(jnp.ones((E,), f32), head,
                                num_segments=N_ENT)
        entity_agg = s / jnp.clip(c, 1.0, None)[:, None]
        item_fusion = itm_t + ent_t[:N_ITEMS]
        pitm = _gather_scatter_call(pad_rows(usr_t, SE), mr_g, mc_s, z_si, SI)
        item_agg_cf = (pitm[:SI] + pitm[SI:])[:N_ITEMS]
        pusr = _gather_scatter_call(pad_rows(item_fusion, SI), mc_g, mr_s,
                                    z_se, SE)
        user_agg = (pusr[:SE] + pusr[SE:])[:N_USERS]
        return entity_agg, user_agg, item_agg_cf

    ent_res, usr_res, itm_res = entity_emb, user_emb, item_emb_cf
    e, u, it = _aggregate(entity_emb, user_emb, item_emb_cf)
    ea, ua, ia = _dr_norm(u, e, it)
    ent_res = ent_res + ea
    usr_res = usr_res + ua
    itm_res = itm_res + ia
    for _ in range(3):
        e, u, it = _aggregate(ea, ua, ia)
        ea, ua, ia = _dr_norm(u, e, it)
    ent_res = ent_res + ea
    usr_res = usr_res + ua
    itm_res = itm_res + ia
    return ent_res, usr_res, itm_res


# speedup vs baseline: 1.0009x; 1.0009x over previous
import functools

import jax
import jax.numpy as jnp
from jax import lax
from jax.experimental import pallas as pl
from jax.experimental.pallas import tpu as pltpu
from jax.experimental.pallas import tpu_sc as plsc

N_ENT = 10000
N_USERS = 10000
N_ITEMS = 5000
E = 320000
NNZ = 500000
D = 128

SE = 10240
SI = 5120
NNZP = 500224
NC, NS = 2, 16
NW = NC * NS
IR = 4
BLK = 512


def _sc_mesh():
    return plsc.VectorSubcoreMesh(
        core_axis_name="c", subcore_axis_name="s",
        num_cores=NC, num_subcores=NS)


def _gather_call(table, idx2d):
    nch = idx2d.shape[0] // IR
    iters = -(-nch // NW)
    n_rows = idx2d.shape[0] * 128
    ch_rows = IR * 128

    @functools.partial(
        pl.kernel,
        out_type=jax.ShapeDtypeStruct((n_rows, D), jnp.float32),
        mesh=_sc_mesh(),
        scratch_types=[
            pltpu.VMEM((IR, 128), jnp.int32),
            pltpu.VMEM((ch_rows, D), jnp.float32),
            pltpu.SemaphoreType.DMA,
        ],
    )
    def k(tab, idx, out, idx_v, rows_v, sem):
        w = lax.axis_index("s") * NC + lax.axis_index("c")

        def body(i, carry):
            ch = w + i * NW

            @pl.when(ch < nch)
            def _():
                pltpu.sync_copy(idx.at[pl.ds(ch * IR, IR)], idx_v)
                descs = [
                    pltpu.async_copy(tab.at[idx_v.at[j]],
                                     rows_v.at[pl.ds(j * 128, 128)], sem)
                    for j in range(IR)
                ]
                for d in descs:
                    d.wait()
                pltpu.sync_copy(rows_v, out.at[pl.ds(ch * ch_rows, ch_rows)])

            return carry

        lax.fori_loop(0, iters, body, 0)

    return k(table, idx2d)


def _gather_scatter_call(table, gidx2d, sidx2d, zeros, S, ir=2):
    ch_rows = ir * 128
    nch = gidx2d.shape[0] // ir
    iters = -(-nch // NW)
    rpt = S // NS

    @functools.partial(
        pl.kernel,
        out_type=jax.ShapeDtypeStruct((NC * S, D), jnp.float32),
        mesh=_sc_mesh(),
        scratch_types=[
            pltpu.VMEM((ir, 128), jnp.int32),
            pltpu.VMEM((ir, 128), jnp.int32),
            pltpu.VMEM((ch_rows, D), jnp.float32),
            pltpu.VMEM_SHARED((S, D), jnp.float32),
            pltpu.SemaphoreType.DMA,
        ],
    )
    def k(tab, gidx, sidx, zz, out, gi_v, si_v, rows_v, acc, sem):
        c = lax.axis_index("c")
        s = lax.axis_index("s")
        w = s * NC + c
        pltpu.sync_copy(zz.at[pl.ds(s * rpt, rpt)], acc.at[pl.ds(s * rpt, rpt)])
        plsc.subcore_barrier()

        def body(i, carry):
            ch = w + i * NW

            @pl.when(ch < nch)
            def _():
                pltpu.sync_copy(gidx.at[pl.ds(ch * ir, ir)], gi_v)
                pltpu.sync_copy(sidx.at[pl.ds(ch * ir, ir)], si_v)
                descs = [
                    pltpu.async_copy(tab.at[gi_v.at[j]],
                                     rows_v.at[pl.ds(j * 128, 128)], sem)
                    for j in range(ir)
                ]
                for d in descs:
                    d.wait()
                for j in range(ir):
                    pltpu.sync_copy(rows_v.at[pl.ds(j * 128, 128)],
                                    acc.at[si_v.at[j]], add=True)

            return carry

        lax.fori_loop(0, iters, body, 0)
        plsc.subcore_barrier()
        pltpu.sync_copy(acc.at[pl.ds(s * rpt, rpt)],
                        out.at[pl.ds(c * S + s * rpt, rpt)])

    return k(table, gidx2d, sidx2d, zeros)


def _mob(x, y, x2, y2, xy):
    num = (1.0 + 2.0 * xy + y2) * x + (1.0 - x2) * y
    den = 1.0 + 2.0 * xy + x2 * y2
    return num / jnp.clip(den, 1e-15, None)


def _edge_body(h_ref, t_ref, et_ref, w_ref, o_ref):
    h = h_ref[...]
    t = t_ref[...]
    W = w_ref[...]
    rel = et_ref[0, 0, :] - 1

    r = jnp.zeros_like(h)
    for k in range(10):
        r = r + jnp.where(rel == k, 1.0, 0.0)[:, None] * W[k][None, :]

    def rsum(x, y):
        p = x * y
        v = p[:, 0:8]
        for g in range(1, 16):
            v = v + p[:, 8 * g:8 * g + 8]
        v = v[:, 0:4] + v[:, 4:8]
        v = v[:, 0:2] + v[:, 2:4]
        return v[:, 0:1] + v[:, 1:2]

    def nrm(x):
        return jnp.clip(jnp.sqrt(rsum(x, x)), 1e-15, None)

    hn = nrm(h)
    hh = jnp.tanh(hn) * h / hn
    hh2 = rsum(hh, hh)
    lam = 2.0 / jnp.clip(1.0 - hh2, 1e-15, None)

    tn = nrm(t)
    u = jnp.tanh(lam * tn / 2.0) * t / tn
    ht = _mob(hh, u, hh2, rsum(u, u), rsum(hh, u))

    rn = nrm(r)
    v = jnp.tanh(lam * rn / 2.0) * r / rn
    hr = _mob(hh, v, hh2, rsum(v, v), rsum(hh, v))

    a2 = rsum(ht, ht)
    b2 = rsum(hr, hr)
    ab = rsum(ht, hr)
    num_r = (1.0 + 2.0 * ab + b2) * ht + (1.0 - a2) * hr
    den_r = jnp.clip(1.0 + 2.0 * ab + a2 * b2, 1e-15, None)
    res = num_r / den_r
    n = nrm(res)
    mx = 1.0 - 1e-5
    res = jnp.where(n > mx, num_r / (den_r * n) * mx, res)

    sub = _mob(-hh, res, hh2, rsum(res, res), rsum(-hh, res))
    sn = nrm(sub)
    z = jnp.clip(sn, -1.0 + 1e-7, 1.0 - 1e-7)
    art = 0.5 * (jnp.log1p(z) - jnp.log1p(-z))
    tl = jnp.clip(1.0 - hh2, 1e-15, None)
    o_ref[...] = (tl * art) * sub / sn


def _edge_call(htr, et3, w_pad):
    nb = E // BLK
    return pl.pallas_call(
        _edge_body,
        grid=(nb,),
        in_specs=[
            pl.BlockSpec((BLK, D), lambda i: (i, 0)),
            pl.BlockSpec((BLK, D), lambda i: (i + nb, 0)),
            pl.BlockSpec((1, 1, BLK), lambda i: (i, 0, 0)),
            pl.BlockSpec((16, D), lambda i: (0, 0)),
        ],
        out_specs=pl.BlockSpec((BLK, D), lambda i: (i, 0)),
        out_shape=jax.ShapeDtypeStruct((E, D), jnp.float32),
    )(htr, htr, et3, w_pad)


def _l2n(x):
    return x / jnp.clip(jnp.linalg.norm(x, axis=-1, keepdims=True), 1e-12, None)


def _dr_norm(user_emb, entity_emb, item_emb_cf):
    for i in range(3):
        entity_emb = _l2n(entity_emb)
        user_emb = _l2n(user_emb)
        item_emb_cf = _l2n(item_emb_cf)
        if i == 0:
            ea, ua, ia = entity_emb, user_emb, item_emb_cf
        else:
            ea = ea + entity_emb
            ua = ua + user_emb
            ia = ia + item_emb_cf
    return ea, ua, ia


def kernel(user_emb, entity_emb, item_emb_cf, edge_index, edge_type,
           interact_indices, interact_values, relation_weight):
    f32 = jnp.float32
    ei = edge_index.astype(jnp.int32)
    head = ei[0]
    tail = ei[1]
    ht_idx = jnp.concatenate([head, tail]).reshape(2 * E // 128, 128)
    et3 = edge_type.astype(jnp.int32).reshape(E // BLK, 1, BLK)

    mr = interact_indices[0].astype(jnp.int32)
    mc = interact_indices[1].astype(jnp.int32)
    pad = NNZP - NNZ
    mr_g = jnp.concatenate([mr, jnp.zeros((pad,), jnp.int32)]).reshape(-1, 128)
    mc_g = jnp.concatenate([mc, jnp.zeros((pad,), jnp.int32)]).reshape(-1, 128)
    mr_s = jnp.concatenate(
        [mr, jnp.full((pad,), N_USERS, jnp.int32)]).reshape(-1, 128)
    mc_s = jnp.concatenate(
        [mc, jnp.full((pad,), N_ITEMS, jnp.int32)]).reshape(-1, 128)

    w_pad = jnp.zeros((16, D), f32).at[:relation_weight.shape[0]].set(
        relation_weight)
    z_se = jnp.zeros((SE, D), f32)
    z_si = jnp.zeros((SI, D), f32)

    def pad_rows(x, S):
        return jnp.zeros((S, D), f32).at[:x.shape[0]].set(x)

    def _aggregate(ent_t, usr_t, itm_t):
        htr = _gather_call(pad_rows(ent_t, SE), ht_idx)
        res = _edge_call(htr, et3, w_pad)
        s = jax.ops.segment_sum(res, head, num_segments=N_ENT)
        c = jax.ops.segment_sum(jnp.ones((E,), f32), head,
                                num_segments=N_ENT)
        entity_agg = s / jnp.clip(c, 1.0, None)[:, None]
        item_fusion = itm_t + ent_t[:N_ITEMS]
        pitm = _gather_scatter_call(pad_rows(usr_t, SE), mr_g, mc_s, z_si, SI,
                                    ir=4)
        item_agg_cf = (pitm[:SI] + pitm[SI:])[:N_ITEMS]
        pusr = _gather_scatter_call(pad_rows(item_fusion, SI), mc_g, mr_s,
                                    z_se, SE)
        user_agg = (pusr[:SE] + pusr[SE:])[:N_USERS]
        return entity_agg, user_agg, item_agg_cf

    ent_res, usr_res, itm_res = entity_emb, user_emb, item_emb_cf
    e, u, it = _aggregate(entity_emb, user_emb, item_emb_cf)
    ea, ua, ia = _dr_norm(u, e, it)
    ent_res = ent_res + ea
    usr_res = usr_res + ua
    itm_res = itm_res + ia
    for _ in range(3):
        e, u, it = _aggregate(ea, ua, ia)
        ea, ua, ia = _dr_norm(u, e, it)
    ent_res = ent_res + ea
    usr_res = usr_res + ua
    itm_res = itm_res + ia
    return ent_res, usr_res, itm_res
